# Initial kernel scaffold; baseline (speedup 1.0000x reference)
import jax
import jax.numpy as jnp
from jax.experimental import pallas as pl


def kernel(cat_idx, table):
    raise NotImplementedError("write your pallas kernel here")


# speedup vs baseline: 1.0946x; 1.0946x over previous
import functools

import jax
import jax.numpy as jnp
from jax import lax
from jax.experimental import pallas as pl
from jax.experimental.pallas import tpu as pltpu
from jax.experimental.pallas import tpu_sc as plsc

EMBED_DIM = 32
NUM_CORES = 2
NUM_SUBCORES = 16
NUM_WORKERS = NUM_CORES * NUM_SUBCORES
CHUNK = 1024


def kernel(cat_idx, table):
    batch, seq = cat_idx.shape
    n = batch * seq
    per_worker = n // NUM_WORKERS
    idx = cat_idx.reshape(n).astype(jnp.int32)

    mesh = plsc.VectorSubcoreMesh(core_axis_name="c", subcore_axis_name="s")

    @functools.partial(
        pl.kernel,
        out_type=jax.ShapeDtypeStruct((n, EMBED_DIM), table.dtype),
        mesh=mesh,
        scratch_types=[
            pltpu.VMEM((CHUNK,), jnp.int32),
            pltpu.VMEM((CHUNK, EMBED_DIM), jnp.float32),
            pltpu.SemaphoreType.DMA,
        ],
        compiler_params=pltpu.CompilerParams(use_tc_tiling_on_sc=False),
    )
    def gather_kernel(tbl_hbm, idx_hbm, out_hbm, idx_v, rows_v, sem):
        wid = lax.axis_index("s") * NUM_CORES + lax.axis_index("c")
        base = wid * per_worker

        @pl.loop(0, per_worker, step=CHUNK)
        def _(off):
            pltpu.sync_copy(idx_hbm.at[pl.ds(base + off, CHUNK)], idx_v)
            pltpu.async_copy(tbl_hbm.at[idx_v], rows_v, sem).wait()
            pltpu.sync_copy(rows_v, out_hbm.at[pl.ds(base + off, CHUNK)])

    out = gather_kernel(table, idx)
    return out.reshape(batch, seq, EMBED_DIM)


# speedup vs baseline: 1.1087x; 1.0129x over previous
import functools

import jax
import jax.numpy as jnp
from jax import lax
from jax.experimental import pallas as pl
from jax.experimental.pallas import tpu as pltpu
from jax.experimental.pallas import tpu_sc as plsc

EMBED_DIM = 32
NUM_CORES = 2
NUM_SUBCORES = 16
NUM_WORKERS = NUM_CORES * NUM_SUBCORES
CHUNK = 1600


def kernel(cat_idx, table):
    batch, seq = cat_idx.shape
    n = batch * seq
    per_worker = n // NUM_WORKERS
    n_chunks = per_worker // CHUNK
    idx = cat_idx.reshape(n).astype(jnp.int32)

    mesh = plsc.VectorSubcoreMesh(core_axis_name="c", subcore_axis_name="s")

    @functools.partial(
        pl.kernel,
        out_type=jax.ShapeDtypeStruct((n, EMBED_DIM), table.dtype),
        mesh=mesh,
        scratch_types=[
            pltpu.VMEM((CHUNK,), jnp.int32),
            pltpu.VMEM((CHUNK,), jnp.int32),
            pltpu.VMEM((CHUNK, EMBED_DIM), jnp.float32),
            pltpu.VMEM((CHUNK, EMBED_DIM), jnp.float32),
            pltpu.SemaphoreType.DMA,
            pltpu.SemaphoreType.DMA,
            pltpu.SemaphoreType.DMA,
            pltpu.SemaphoreType.DMA,
        ],
        compiler_params=pltpu.CompilerParams(use_tc_tiling_on_sc=False),
    )
    def gather_kernel(tbl_hbm, idx_hbm, out_hbm,
                      idx_v0, idx_v1, rows_v0, rows_v1,
                      gsem0, gsem1, osem0, osem1):
        wid = lax.axis_index("s") * NUM_CORES + lax.axis_index("c")
        base = wid * per_worker
        idx_v = (idx_v0, idx_v1)
        rows_v = (rows_v0, rows_v1)
        gsem = (gsem0, gsem1)
        osem = (osem0, osem1)

        def start_gather(c):
            b = c % 2
            pltpu.sync_copy(idx_hbm.at[pl.ds(base + c * CHUNK, CHUNK)], idx_v[b])
            return pltpu.async_copy(tbl_hbm.at[idx_v[b]], rows_v[b], gsem[b])

        gathers = [None, None]
        outs = [None, None]
        gathers[0] = start_gather(0)
        for c in range(n_chunks):
            b = c % 2
            nb = 1 - b
            if c + 1 < n_chunks:
                if outs[nb] is not None:
                    outs[nb].wait()
                gathers[nb] = start_gather(c + 1)
            gathers[b].wait()
            outs[b] = pltpu.async_copy(
                rows_v[b], out_hbm.at[pl.ds(base + c * CHUNK, CHUNK)], osem[b])
        for o in outs:
            if o is not None:
                o.wait()

    out = gather_kernel(table, idx)
    return out.reshape(batch, seq, EMBED_DIM)


# speedup vs baseline: 1.4407x; 1.2994x over previous
import functools

import jax
import jax.numpy as jnp
from jax import lax
from jax.experimental import pallas as pl
from jax.experimental.pallas import tpu as pltpu
from jax.experimental.pallas import tpu_sc as plsc

EMBED_DIM = 32
NUM_CORES = 2
NUM_SUBCORES = 16
NUM_WORKERS = NUM_CORES * NUM_SUBCORES
CHUNK = 1600
TBL_BLK = 8192
OUT_BLK = 2048


def _tbl_t_body(x_ref, o_ref):
    o_ref[...] = x_ref[...].T


def _out_t_body(x_ref, o_ref):
    o_ref[...] = x_ref[...].T


def _sc_gather(table_rm, idx_flat, n):
    per_worker = n // NUM_WORKERS
    n_chunks = per_worker // CHUNK
    mesh = plsc.VectorSubcoreMesh(core_axis_name="c", subcore_axis_name="s")

    @functools.partial(
        pl.kernel,
        out_type=jax.ShapeDtypeStruct((n, EMBED_DIM), table_rm.dtype),
        mesh=mesh,
        scratch_types=[
            pltpu.VMEM((CHUNK,), jnp.int32),
            pltpu.VMEM((CHUNK,), jnp.int32),
            pltpu.VMEM((CHUNK, EMBED_DIM), jnp.float32),
            pltpu.VMEM((CHUNK, EMBED_DIM), jnp.float32),
            pltpu.SemaphoreType.DMA,
            pltpu.SemaphoreType.DMA,
            pltpu.SemaphoreType.DMA,
            pltpu.SemaphoreType.DMA,
        ],
        compiler_params=pltpu.CompilerParams(use_tc_tiling_on_sc=False),
    )
    def gather_kernel(tbl_hbm, idx_hbm, out_hbm,
                      idx_v0, idx_v1, rows_v0, rows_v1,
                      gsem0, gsem1, osem0, osem1):
        wid = lax.axis_index("s") * NUM_CORES + lax.axis_index("c")
        base = wid * per_worker
        idx_v = (idx_v0, idx_v1)
        rows_v = (rows_v0, rows_v1)
        gsem = (gsem0, gsem1)
        osem = (osem0, osem1)

        def start_gather(c):
            b = c % 2
            pltpu.sync_copy(idx_hbm.at[pl.ds(base + c * CHUNK, CHUNK)], idx_v[b])
            return pltpu.async_copy(tbl_hbm.at[idx_v[b]], rows_v[b], gsem[b])

        gathers = [None, None]
        outs = [None, None]
        gathers[0] = start_gather(0)
        for c in range(n_chunks):
            b = c % 2
            nb = 1 - b
            if c + 1 < n_chunks:
                if outs[nb] is not None:
                    outs[nb].wait()
                gathers[nb] = start_gather(c + 1)
            gathers[b].wait()
            outs[b] = pltpu.async_copy(
                rows_v[b], out_hbm.at[pl.ds(base + c * CHUNK, CHUNK)], osem[b])
        for o in outs:
            if o is not None:
                o.wait()

    return gather_kernel(table_rm, idx_flat)


def kernel(cat_idx, table):
    batch, seq = cat_idx.shape
    n = batch * seq
    nv = table.shape[0]

    idx_flat = jnp.swapaxes(cat_idx, 0, 1).reshape(n).astype(jnp.int32)

    table_t = jnp.swapaxes(table, 0, 1)
    n_tb = (nv + TBL_BLK - 1) // TBL_BLK
    table_rm = pl.pallas_call(
        _tbl_t_body,
        grid=(n_tb,),
        in_specs=[pl.BlockSpec((EMBED_DIM, TBL_BLK), lambda k: (0, k))],
        out_specs=pl.BlockSpec((TBL_BLK, EMBED_DIM), lambda k: (k, 0)),
        out_shape=jax.ShapeDtypeStruct((nv, EMBED_DIM), table.dtype),
        compiler_params=pltpu.CompilerParams(
            dimension_semantics=("parallel",)),
    )(table_t)

    lin = _sc_gather(table_rm, idx_flat, n)

    n_ob = batch // OUT_BLK
    phys2d = pl.pallas_call(
        _out_t_body,
        grid=(seq, n_ob),
        in_specs=[pl.BlockSpec((OUT_BLK, EMBED_DIM),
                               lambda j, i: (j * n_ob + i, 0))],
        out_specs=pl.BlockSpec((EMBED_DIM, OUT_BLK), lambda j, i: (j, i)),
        out_shape=jax.ShapeDtypeStruct((seq * EMBED_DIM, batch), table.dtype),
        compiler_params=pltpu.CompilerParams(
            dimension_semantics=("parallel", "parallel")),
    )(lin)

    phys = phys2d.reshape(seq, EMBED_DIM, batch)
    return jnp.transpose(phys, (2, 0, 1))


# speedup vs baseline: 1.9606x; 1.3609x over previous
import functools

import jax
import jax.numpy as jnp
from jax import lax
from jax.experimental import pallas as pl
from jax.experimental.pallas import tpu as pltpu
from jax.experimental.pallas import tpu_sc as plsc

EMBED_DIM = 32
NUM_CORES = 2
NUM_SUBCORES = 16
NUM_WORKERS = NUM_CORES * NUM_SUBCORES
CHUNK = 1600
TBL_BLK = 8192
OUT_BLK = 2048


def _tbl_t_body(x_ref, o_ref):
    y = x_ref[...].T.reshape(TBL_BLK // 4, 4, EMBED_DIM)
    o_ref[...] = jnp.concatenate([y[:, k, :] for k in range(4)], axis=1)


def _out_t_body(x_ref, o_ref):
    x = x_ref[...]
    z = jnp.concatenate(
        [x[:, k * EMBED_DIM:(k + 1) * EMBED_DIM] for k in range(4)], axis=0)
    o_ref[...] = z.T


def _sc_gather(table_rm, idx_flat, n):
    per_worker = n // NUM_WORKERS
    n_chunks = per_worker // CHUNK
    mesh = plsc.VectorSubcoreMesh(core_axis_name="c", subcore_axis_name="s")

    @functools.partial(
        pl.kernel,
        out_type=jax.ShapeDtypeStruct((n, EMBED_DIM), table_rm.dtype),
        mesh=mesh,
        scratch_types=[
            pltpu.VMEM((CHUNK,), jnp.int32),
            pltpu.VMEM((CHUNK,), jnp.int32),
            pltpu.VMEM((CHUNK, EMBED_DIM), jnp.float32),
            pltpu.VMEM((CHUNK, EMBED_DIM), jnp.float32),
            pltpu.SemaphoreType.DMA,
            pltpu.SemaphoreType.DMA,
            pltpu.SemaphoreType.DMA,
            pltpu.SemaphoreType.DMA,
        ],
        compiler_params=pltpu.CompilerParams(use_tc_tiling_on_sc=False),
    )
    def gather_kernel(tbl_hbm, idx_hbm, out_hbm,
                      idx_v0, idx_v1, rows_v0, rows_v1,
                      gsem0, gsem1, osem0, osem1):
        wid = lax.axis_index("s") * NUM_CORES + lax.axis_index("c")
        base = wid * per_worker
        idx_v = (idx_v0, idx_v1)
        rows_v = (rows_v0, rows_v1)
        gsem = (gsem0, gsem1)
        osem = (osem0, osem1)

        def start_gather(c):
            b = c % 2
            pltpu.sync_copy(idx_hbm.at[pl.ds(base + c * CHUNK, CHUNK)], idx_v[b])
            return pltpu.async_copy(tbl_hbm.at[idx_v[b]], rows_v[b], gsem[b])

        gathers = [None, None]
        outs = [None, None]
        gathers[0] = start_gather(0)
        for c in range(n_chunks):
            b = c % 2
            nb = 1 - b
            if c + 1 < n_chunks:
                if outs[nb] is not None:
                    outs[nb].wait()
                gathers[nb] = start_gather(c + 1)
            gathers[b].wait()
            outs[b] = pltpu.async_copy(
                rows_v[b], out_hbm.at[pl.ds(base + c * CHUNK, CHUNK)], osem[b])
        for o in outs:
            if o is not None:
                o.wait()

    return gather_kernel(table_rm, idx_flat)


def kernel(cat_idx, table):
    batch, seq = cat_idx.shape
    n = batch * seq
    nv = table.shape[0]

    n_ob = batch // OUT_BLK
    idx_flat = (
        jnp.swapaxes(cat_idx, 0, 1)
        .reshape(seq, n_ob, 4, OUT_BLK // 4)
        .transpose(0, 1, 3, 2)
        .reshape(n)
        .astype(jnp.int32)
    )

    table_t = jnp.swapaxes(table, 0, 1)
    n_tb = (nv + TBL_BLK - 1) // TBL_BLK
    table_rm4 = pl.pallas_call(
        _tbl_t_body,
        grid=(n_tb,),
        in_specs=[pl.BlockSpec((EMBED_DIM, TBL_BLK), lambda k: (0, k))],
        out_specs=pl.BlockSpec((TBL_BLK // 4, 4 * EMBED_DIM),
                               lambda k: (k, 0)),
        out_shape=jax.ShapeDtypeStruct((nv // 4, 4 * EMBED_DIM), table.dtype),
        compiler_params=pltpu.CompilerParams(
            dimension_semantics=("parallel",)),
    )(table_t)
    table_rm = table_rm4.reshape(nv, EMBED_DIM)

    lin = _sc_gather(table_rm, idx_flat, n)

    lin128 = lin.reshape(n // 4, 4 * EMBED_DIM)
    phys2d = pl.pallas_call(
        _out_t_body,
        grid=(seq, n_ob),
        in_specs=[pl.BlockSpec((OUT_BLK // 4, 4 * EMBED_DIM),
                               lambda j, i: (j * n_ob + i, 0))],
        out_specs=pl.BlockSpec((EMBED_DIM, OUT_BLK), lambda j, i: (j, i)),
        out_shape=jax.ShapeDtypeStruct((seq * EMBED_DIM, batch), table.dtype),
        compiler_params=pltpu.CompilerParams(
            dimension_semantics=("parallel", "parallel")),
    )(lin128)

    phys = phys2d.reshape(seq, EMBED_DIM, batch)
    return jnp.transpose(phys, (2, 0, 1))


# speedup vs baseline: 2.9045x; 1.4815x over previous
import functools

import jax
import jax.numpy as jnp
from jax import lax
from jax.experimental import pallas as pl
from jax.experimental.pallas import tpu as pltpu
from jax.experimental.pallas import tpu_sc as plsc

EMBED_DIM = 32
NUM_CORES = 2
NUM_SUBCORES = 16
NUM_WORKERS = NUM_CORES * NUM_SUBCORES
CHUNK = 1600
TBL_BLK = 8192
OUT_BLK = 2048


def _tbl_t_body(x_ref, o_ref):
    x = x_ref[...]
    b4 = TBL_BLK // 4
    y = jnp.concatenate([x[:, m * b4:(m + 1) * b4] for m in range(4)], axis=0)
    o_ref[...] = y.T


def _out_t_body(x_ref, o_ref):
    xt = x_ref[...].T
    o_ref[...] = jnp.concatenate(
        [xt[k * EMBED_DIM:(k + 1) * EMBED_DIM, :] for k in range(4)], axis=1)


def _sc_gather(table_rm, idx_flat, n):
    per_worker = n // NUM_WORKERS
    n_chunks = per_worker // CHUNK
    mesh = plsc.VectorSubcoreMesh(core_axis_name="c", subcore_axis_name="s")

    @functools.partial(
        pl.kernel,
        out_type=jax.ShapeDtypeStruct((n, EMBED_DIM), table_rm.dtype),
        mesh=mesh,
        scratch_types=[
            pltpu.VMEM((CHUNK,), jnp.int32),
            pltpu.VMEM((CHUNK,), jnp.int32),
            pltpu.VMEM((CHUNK, EMBED_DIM), jnp.float32),
            pltpu.VMEM((CHUNK, EMBED_DIM), jnp.float32),
            pltpu.SemaphoreType.DMA,
            pltpu.SemaphoreType.DMA,
            pltpu.SemaphoreType.DMA,
            pltpu.SemaphoreType.DMA,
        ],
        compiler_params=pltpu.CompilerParams(use_tc_tiling_on_sc=False),
    )
    def gather_kernel(tbl_hbm, idx_hbm, out_hbm,
                      idx_v0, idx_v1, rows_v0, rows_v1,
                      gsem0, gsem1, osem0, osem1):
        wid = lax.axis_index("s") * NUM_CORES + lax.axis_index("c")
        base = wid * per_worker
        idx_v = (idx_v0, idx_v1)
        rows_v = (rows_v0, rows_v1)
        gsem = (gsem0, gsem1)
        osem = (osem0, osem1)

        def start_gather(c):
            b = c % 2
            pltpu.sync_copy(idx_hbm.at[pl.ds(base + c * CHUNK, CHUNK)], idx_v[b])
            return pltpu.async_copy(tbl_hbm.at[idx_v[b]], rows_v[b], gsem[b])

        gathers = [None, None]
        outs = [None, None]
        gathers[0] = start_gather(0)
        for c in range(n_chunks):
            b = c % 2
            nb = 1 - b
            if c + 1 < n_chunks:
                if outs[nb] is not None:
                    outs[nb].wait()
                gathers[nb] = start_gather(c + 1)
            gathers[b].wait()
            outs[b] = pltpu.async_copy(
                rows_v[b], out_hbm.at[pl.ds(base + c * CHUNK, CHUNK)], osem[b])
        for o in outs:
            if o is not None:
                o.wait()

    return gather_kernel(table_rm, idx_flat)


def kernel(cat_idx, table):
    batch, seq = cat_idx.shape
    n = batch * seq
    nv = table.shape[0]
    tb4 = TBL_BLK // 4
    n_ob = batch // OUT_BLK

    v = jnp.swapaxes(cat_idx, 0, 1).astype(jnp.int32)
    rem = v % TBL_BLK
    v = (v - rem) + 4 * (rem % tb4) + rem // tb4
    idx_flat = (
        v.reshape(seq, n_ob, 4, OUT_BLK // 4)
        .transpose(0, 1, 3, 2)
        .reshape(n)
    )

    table_t = jnp.swapaxes(table, 0, 1)
    n_tb = (nv + TBL_BLK - 1) // TBL_BLK
    nv_pad = n_tb * TBL_BLK
    table_rm4 = pl.pallas_call(
        _tbl_t_body,
        grid=(n_tb,),
        in_specs=[pl.BlockSpec((EMBED_DIM, TBL_BLK), lambda k: (0, k))],
        out_specs=pl.BlockSpec((tb4, 4 * EMBED_DIM), lambda k: (k, 0)),
        out_shape=jax.ShapeDtypeStruct((nv_pad // 4, 4 * EMBED_DIM),
                                       table.dtype),
        compiler_params=pltpu.CompilerParams(
            dimension_semantics=("parallel",)),
    )(table_t)
    table_rm = table_rm4.reshape(nv_pad, EMBED_DIM)

    lin = _sc_gather(table_rm, idx_flat, n)

    lin128 = lin.reshape(n // 4, 4 * EMBED_DIM)
    phys2d = pl.pallas_call(
        _out_t_body,
        grid=(seq, n_ob),
        in_specs=[pl.BlockSpec((OUT_BLK // 4, 4 * EMBED_DIM),
                               lambda j, i: (j * n_ob + i, 0))],
        out_specs=pl.BlockSpec((EMBED_DIM, OUT_BLK), lambda j, i: (j, i)),
        out_shape=jax.ShapeDtypeStruct((seq * EMBED_DIM, batch), table.dtype),
        compiler_params=pltpu.CompilerParams(
            dimension_semantics=("parallel", "parallel")),
    )(lin128)

    phys = phys2d.reshape(seq, EMBED_DIM, batch)
    return jnp.transpose(phys, (2, 0, 1))


# speedup vs baseline: 3.6025x; 1.2403x over previous
import functools

import jax
import jax.numpy as jnp
from jax import lax
from jax.experimental import pallas as pl
from jax.experimental.pallas import tpu as pltpu
from jax.experimental.pallas import tpu_sc as plsc

EMBED_DIM = 32
NUM_CORES = 2
NUM_SUBCORES = 16
NUM_WORKERS = NUM_CORES * NUM_SUBCORES
CHUNK = 1024
TBL_BLK = 8192
OUT_BLK = 2048


def _idx_body(x_ref, o_ref):
    j = pl.program_id(0)
    x = x_ref[pl.ds(j, 1), :][0]
    o_ref[...] = (x & ~8191) + ((x & 2047) << 2) + ((x >> 11) & 3)


def _tbl_t_body(x_ref, o_ref):
    x = x_ref[...]
    b4 = TBL_BLK // 4
    y = jnp.concatenate([x[:, m * b4:(m + 1) * b4] for m in range(4)], axis=0)
    o_ref[...] = y.T


def _out_t_body(x_ref, o_ref):
    xt = x_ref[...].T
    o_ref[...] = jnp.concatenate(
        [xt[k * EMBED_DIM:(k + 1) * EMBED_DIM, :] for k in range(4)], axis=1)


def _sc_gather(table_rm, idx_flat, n):
    per_worker = n // NUM_WORKERS
    n_chunks = per_worker // CHUNK
    half = CHUNK // 2
    mesh = plsc.VectorSubcoreMesh(core_axis_name="c", subcore_axis_name="s")

    @functools.partial(
        pl.kernel,
        out_type=jax.ShapeDtypeStruct((n // 4, 4 * EMBED_DIM),
                                      table_rm.dtype),
        mesh=mesh,
        scratch_types=[
            pltpu.VMEM((CHUNK,), jnp.int32),
            pltpu.VMEM((CHUNK,), jnp.int32),
            pltpu.VMEM((CHUNK, EMBED_DIM), jnp.float32),
            pltpu.VMEM((CHUNK, EMBED_DIM), jnp.float32),
            pltpu.SemaphoreType.DMA,
            pltpu.SemaphoreType.DMA,
            pltpu.SemaphoreType.DMA,
            pltpu.SemaphoreType.DMA,
        ],
        compiler_params=pltpu.CompilerParams(use_tc_tiling_on_sc=False),
    )
    def gather_kernel(tbl_hbm, idx_hbm, out_hbm,
                      idx_v0, idx_v1, rows_v0, rows_v1,
                      gsem0, gsem1, osem0, osem1):
        wid = lax.axis_index("s") * NUM_CORES + lax.axis_index("c")
        base = wid * per_worker
        idx_v = (idx_v0, idx_v1)
        rows_v = (rows_v0, rows_v1)
        gsem = (gsem0, gsem1)
        osem = (osem0, osem1)

        def start_gather(c):
            b = c % 2
            pltpu.sync_copy(idx_hbm.at[pl.ds(base + c * CHUNK, CHUNK)], idx_v[b])
            return pltpu.async_copy(tbl_hbm.at[idx_v[b]], rows_v[b], gsem[b])

        def start_drain(c):
            b = c % 2
            copies = []
            for h in range(2):
                p0 = base + c * CHUNK + h * half
                g = p0 // (4 * half)
                u = (p0 // half) % 4
                copies.append(pltpu.async_copy(
                    rows_v[b].at[pl.ds(h * half, half)],
                    out_hbm.at[pl.ds(g * half, half),
                               pl.ds(u * EMBED_DIM, EMBED_DIM)],
                    osem[b]))
            return copies

        gathers = [None, None]
        outs = [None, None]
        gathers[0] = start_gather(0)
        for c in range(n_chunks):
            b = c % 2
            nb = 1 - b
            if c + 1 < n_chunks:
                if outs[nb] is not None:
                    for o in outs[nb]:
                        o.wait()
                gathers[nb] = start_gather(c + 1)
            gathers[b].wait()
            outs[b] = start_drain(c)
        for pair in outs:
            if pair is not None:
                for o in pair:
                    o.wait()

    return gather_kernel(table_rm, idx_flat)


def kernel(cat_idx, table):
    batch, seq = cat_idx.shape
    n = batch * seq
    nv = table.shape[0]
    tb4 = TBL_BLK // 4
    n_ob = batch // OUT_BLK
    half_seq = seq // 2

    idx_t = jnp.swapaxes(cat_idx, 0, 1).astype(jnp.int32)
    idx_flat = pl.pallas_call(
        _idx_body,
        grid=(seq,),
        in_specs=[pl.BlockSpec((seq, batch), lambda j: (0, 0))],
        out_specs=pl.BlockSpec((batch,), lambda j: (j,)),
        out_shape=jax.ShapeDtypeStruct((n,), jnp.int32),
        compiler_params=pltpu.CompilerParams(
            dimension_semantics=("arbitrary",)),
    )(idx_t)

    table_t = jnp.swapaxes(table, 0, 1)
    n_tb = (nv + TBL_BLK - 1) // TBL_BLK
    nv_pad = n_tb * TBL_BLK
    table_rm4 = pl.pallas_call(
        _tbl_t_body,
        grid=(n_tb,),
        in_specs=[pl.BlockSpec((EMBED_DIM, TBL_BLK), lambda k: (0, k))],
        out_specs=pl.BlockSpec((tb4, 4 * EMBED_DIM), lambda k: (k, 0)),
        out_shape=jax.ShapeDtypeStruct((nv_pad // 4, 4 * EMBED_DIM),
                                       table.dtype),
        compiler_params=pltpu.CompilerParams(
            dimension_semantics=("parallel",)),
    )(table_t)
    table_rm = table_rm4.reshape(nv_pad, EMBED_DIM)

    lin128 = _sc_gather(table_rm, idx_flat, n)

    phys2d = pl.pallas_call(
        _out_t_body,
        grid=(seq, n_ob),
        in_specs=[pl.BlockSpec(
            (OUT_BLK // 4, 4 * EMBED_DIM),
            lambda j, i: (j * n_ob + i, 0))],
        out_specs=pl.BlockSpec((EMBED_DIM, OUT_BLK), lambda j, i: (j, i)),
        out_shape=jax.ShapeDtypeStruct((seq * EMBED_DIM, batch), table.dtype),
        compiler_params=pltpu.CompilerParams(
            dimension_semantics=("parallel", "parallel")),
    )(lin128)

    phys = phys2d.reshape(seq, EMBED_DIM, batch)
    return jnp.transpose(phys, (2, 0, 1))


# speedup vs baseline: 5.6159x; 1.5589x over previous
import functools

import jax
import jax.numpy as jnp
from jax import lax
from jax.experimental import pallas as pl
from jax.experimental.pallas import tpu as pltpu
from jax.experimental.pallas import tpu_sc as plsc

EMBED_DIM = 32
NUM_CORES = 2
NUM_SUBCORES = 16
NUM_WORKERS = NUM_CORES * NUM_SUBCORES
CHUNK = 1024
TBL_BLK = 16384
OUT_BLK = 8192


def _idx_body(x_ref, o_ref):
    j = pl.program_id(0)
    x = x_ref[pl.ds(j, 1), :][0]
    tb4 = TBL_BLK // 4
    o_ref[...] = (x & ~(TBL_BLK - 1)) + ((x & (tb4 - 1)) << 2) + (
        (x >> 12) & 3)


def _tbl_t_body(x_ref, o_ref):
    x = x_ref[...]
    b4 = TBL_BLK // 4
    y = jnp.concatenate([x[:, m * b4:(m + 1) * b4] for m in range(4)], axis=0)
    o_ref[...] = y.T


def _out_t_body(x_ref, o_ref):
    xt = x_ref[...].T
    ng = OUT_BLK // 2048
    o_ref[...] = jnp.concatenate(
        [xt[k * EMBED_DIM:(k + 1) * EMBED_DIM, g * 512:(g + 1) * 512]
         for g in range(ng) for k in range(4)], axis=1)


def _sc_gather(table_rm, idx_flat, n):
    per_worker = n // NUM_WORKERS
    n_chunks = per_worker // CHUNK
    half = CHUNK // 2
    mesh = plsc.VectorSubcoreMesh(core_axis_name="c", subcore_axis_name="s")

    @functools.partial(
        pl.kernel,
        out_type=jax.ShapeDtypeStruct((n // 4, 4 * EMBED_DIM),
                                      table_rm.dtype),
        mesh=mesh,
        scratch_types=[
            pltpu.VMEM((CHUNK,), jnp.int32),
            pltpu.VMEM((CHUNK,), jnp.int32),
            pltpu.VMEM((CHUNK, EMBED_DIM), jnp.float32),
            pltpu.VMEM((CHUNK, EMBED_DIM), jnp.float32),
            pltpu.SemaphoreType.DMA,
            pltpu.SemaphoreType.DMA,
            pltpu.SemaphoreType.DMA,
            pltpu.SemaphoreType.DMA,
        ],
        compiler_params=pltpu.CompilerParams(use_tc_tiling_on_sc=False),
    )
    def gather_kernel(tbl_hbm, idx_hbm, out_hbm,
                      idx_v0, idx_v1, rows_v0, rows_v1,
                      gsem0, gsem1, osem0, osem1):
        wid = lax.axis_index("s") * NUM_CORES + lax.axis_index("c")
        base = wid * per_worker
        idx_v = (idx_v0, idx_v1)
        rows_v = (rows_v0, rows_v1)
        gsem = (gsem0, gsem1)
        osem = (osem0, osem1)

        def start_gather(c):
            b = c % 2
            pltpu.sync_copy(idx_hbm.at[pl.ds(base + c * CHUNK, CHUNK)], idx_v[b])
            return pltpu.async_copy(tbl_hbm.at[idx_v[b]], rows_v[b], gsem[b])

        def start_drain(c):
            b = c % 2
            copies = []
            for h in range(2):
                p0 = base + c * CHUNK + h * half
                g = p0 // (4 * half)
                u = (p0 // half) % 4
                copies.append(pltpu.async_copy(
                    rows_v[b].at[pl.ds(h * half, half)],
                    out_hbm.at[pl.ds(g * half, half),
                               pl.ds(u * EMBED_DIM, EMBED_DIM)],
                    osem[b]))
            return copies

        gathers = [None, None]
        outs = [None, None]
        gathers[0] = start_gather(0)
        for c in range(n_chunks):
            b = c % 2
            nb = 1 - b
            if c + 1 < n_chunks:
                if outs[nb] is not None:
                    for o in outs[nb]:
                        o.wait()
                gathers[nb] = start_gather(c + 1)
            gathers[b].wait()
            outs[b] = start_drain(c)
        for pair in outs:
            if pair is not None:
                for o in pair:
                    o.wait()

    return gather_kernel(table_rm, idx_flat)


def kernel(cat_idx, table):
    batch, seq = cat_idx.shape
    n = batch * seq
    nv = table.shape[0]
    tb4 = TBL_BLK // 4
    n_ob = batch // OUT_BLK
    half_seq = seq // 2

    idx_t = jnp.swapaxes(cat_idx, 0, 1).astype(jnp.int32)
    idx_flat = pl.pallas_call(
        _idx_body,
        grid=(seq,),
        in_specs=[pl.BlockSpec((seq, batch), lambda j: (0, 0))],
        out_specs=pl.BlockSpec((batch,), lambda j: (j,)),
        out_shape=jax.ShapeDtypeStruct((n,), jnp.int32),
        compiler_params=pltpu.CompilerParams(
            dimension_semantics=("arbitrary",)),
    )(idx_t)

    table_t = jnp.swapaxes(table, 0, 1)
    n_tb = (nv + TBL_BLK - 1) // TBL_BLK
    nv_pad = n_tb * TBL_BLK
    table_rm4 = pl.pallas_call(
        _tbl_t_body,
        grid=(n_tb,),
        in_specs=[pl.BlockSpec((EMBED_DIM, TBL_BLK), lambda k: (0, k))],
        out_specs=pl.BlockSpec((tb4, 4 * EMBED_DIM), lambda k: (k, 0)),
        out_shape=jax.ShapeDtypeStruct((nv_pad // 4, 4 * EMBED_DIM),
                                       table.dtype),
        compiler_params=pltpu.CompilerParams(
            dimension_semantics=("parallel",)),
    )(table_t)
    table_rm = table_rm4.reshape(nv_pad, EMBED_DIM)

    lin128 = _sc_gather(table_rm, idx_flat, n)

    phys2d = pl.pallas_call(
        _out_t_body,
        grid=(seq, n_ob),
        in_specs=[pl.BlockSpec(
            (OUT_BLK // 4, 4 * EMBED_DIM),
            lambda j, i: (j * n_ob + i, 0))],
        out_specs=pl.BlockSpec((EMBED_DIM, OUT_BLK), lambda j, i: (j, i)),
        out_shape=jax.ShapeDtypeStruct((seq * EMBED_DIM, batch), table.dtype),
        compiler_params=pltpu.CompilerParams(
            dimension_semantics=("parallel", "parallel")),
    )(lin128)

    phys = phys2d.reshape(seq, EMBED_DIM, batch)
    return jnp.transpose(phys, (2, 0, 1))


# speedup vs baseline: 6.4787x; 1.1536x over previous
import functools

import jax
import jax.numpy as jnp
from jax import lax
from jax.experimental import pallas as pl
from jax.experimental.pallas import tpu as pltpu
from jax.experimental.pallas import tpu_sc as plsc

EMBED_DIM = 32
NUM_CORES = 2
NUM_SUBCORES = 16
NUM_WORKERS = NUM_CORES * NUM_SUBCORES
CHUNK = 1024
TBL_BLK = 32768
OUT_BLK = 16384


def _idx_body(x_ref, o_ref):
    j = pl.program_id(0)
    x = x_ref[pl.ds(j, 1), :][0]
    tb4 = TBL_BLK // 4
    sh = tb4.bit_length() - 1
    o_ref[...] = (x & ~(TBL_BLK - 1)) + ((x & (tb4 - 1)) << 2) + (
        (x >> sh) & 3)


def _tbl_t_body(x_ref, o_ref):
    x = x_ref[...]
    b4 = TBL_BLK // 4
    y = jnp.concatenate([x[:, m * b4:(m + 1) * b4] for m in range(4)], axis=0)
    o_ref[...] = y.T


def _out_t_body(x_ref, o_ref):
    xt = x_ref[...].T
    ng = OUT_BLK // 2048
    o_ref[...] = jnp.concatenate(
        [xt[k * EMBED_DIM:(k + 1) * EMBED_DIM, g * 512:(g + 1) * 512]
         for g in range(ng) for k in range(4)], axis=1)


def _sc_gather(table_rm, idx_flat, n):
    per_worker = n // NUM_WORKERS
    n_chunks = per_worker // CHUNK
    half = CHUNK // 2
    mesh = plsc.VectorSubcoreMesh(core_axis_name="c", subcore_axis_name="s")

    @functools.partial(
        pl.kernel,
        out_type=jax.ShapeDtypeStruct((n // 4, 4 * EMBED_DIM),
                                      table_rm.dtype),
        mesh=mesh,
        scratch_types=[
            pltpu.VMEM((CHUNK,), jnp.int32),
            pltpu.VMEM((CHUNK,), jnp.int32),
            pltpu.VMEM((CHUNK, EMBED_DIM), jnp.float32),
            pltpu.VMEM((CHUNK, EMBED_DIM), jnp.float32),
            pltpu.SemaphoreType.DMA,
            pltpu.SemaphoreType.DMA,
            pltpu.SemaphoreType.DMA,
            pltpu.SemaphoreType.DMA,
        ],
        compiler_params=pltpu.CompilerParams(use_tc_tiling_on_sc=False),
    )
    def gather_kernel(tbl_hbm, idx_hbm, out_hbm,
                      idx_v0, idx_v1, rows_v0, rows_v1,
                      gsem0, gsem1, osem0, osem1):
        wid = lax.axis_index("s") * NUM_CORES + lax.axis_index("c")
        base = wid * per_worker
        idx_v = (idx_v0, idx_v1)
        rows_v = (rows_v0, rows_v1)
        gsem = (gsem0, gsem1)
        osem = (osem0, osem1)

        def start_gather(c):
            b = c % 2
            pltpu.sync_copy(idx_hbm.at[pl.ds(base + c * CHUNK, CHUNK)], idx_v[b])
            return pltpu.async_copy(tbl_hbm.at[idx_v[b]], rows_v[b], gsem[b])

        def start_drain(c):
            b = c % 2
            copies = []
            for h in range(2):
                p0 = base + c * CHUNK + h * half
                g = p0 // (4 * half)
                u = (p0 // half) % 4
                copies.append(pltpu.async_copy(
                    rows_v[b].at[pl.ds(h * half, half)],
                    out_hbm.at[pl.ds(g * half, half),
                               pl.ds(u * EMBED_DIM, EMBED_DIM)],
                    osem[b]))
            return copies

        gathers = [None, None]
        outs = [None, None]
        gathers[0] = start_gather(0)
        for c in range(n_chunks):
            b = c % 2
            nb = 1 - b
            if c + 1 < n_chunks:
                if outs[nb] is not None:
                    for o in outs[nb]:
                        o.wait()
                gathers[nb] = start_gather(c + 1)
            gathers[b].wait()
            outs[b] = start_drain(c)
        for pair in outs:
            if pair is not None:
                for o in pair:
                    o.wait()

    return gather_kernel(table_rm, idx_flat)


def kernel(cat_idx, table):
    batch, seq = cat_idx.shape
    n = batch * seq
    nv = table.shape[0]
    tb4 = TBL_BLK // 4
    n_ob = batch // OUT_BLK
    half_seq = seq // 2

    idx_t = jnp.swapaxes(cat_idx, 0, 1).astype(jnp.int32)
    idx_flat = pl.pallas_call(
        _idx_body,
        grid=(seq,),
        in_specs=[pl.BlockSpec((seq, batch), lambda j: (0, 0))],
        out_specs=pl.BlockSpec((batch,), lambda j: (j,)),
        out_shape=jax.ShapeDtypeStruct((n,), jnp.int32),
        compiler_params=pltpu.CompilerParams(
            dimension_semantics=("arbitrary",)),
    )(idx_t)

    table_t = jnp.swapaxes(table, 0, 1)
    n_tb = (nv + TBL_BLK - 1) // TBL_BLK
    nv_pad = n_tb * TBL_BLK
    table_rm4 = pl.pallas_call(
        _tbl_t_body,
        grid=(n_tb,),
        in_specs=[pl.BlockSpec((EMBED_DIM, TBL_BLK), lambda k: (0, k))],
        out_specs=pl.BlockSpec((tb4, 4 * EMBED_DIM), lambda k: (k, 0)),
        out_shape=jax.ShapeDtypeStruct((nv_pad // 4, 4 * EMBED_DIM),
                                       table.dtype),
        compiler_params=pltpu.CompilerParams(
            dimension_semantics=("parallel",)),
    )(table_t)
    table_rm = table_rm4.reshape(nv_pad, EMBED_DIM)

    lin128 = _sc_gather(table_rm, idx_flat, n)

    phys2d = pl.pallas_call(
        _out_t_body,
        grid=(seq, n_ob),
        in_specs=[pl.BlockSpec(
            (OUT_BLK // 4, 4 * EMBED_DIM),
            lambda j, i: (j * n_ob + i, 0))],
        out_specs=pl.BlockSpec((EMBED_DIM, OUT_BLK), lambda j, i: (j, i)),
        out_shape=jax.ShapeDtypeStruct((seq * EMBED_DIM, batch), table.dtype),
        compiler_params=pltpu.CompilerParams(
            dimension_semantics=("parallel", "parallel")),
    )(lin128)

    phys = phys2d.reshape(seq, EMBED_DIM, batch)
    return jnp.transpose(phys, (2, 0, 1))


# speedup vs baseline: 6.5174x; 1.0060x over previous
import functools

import jax
import jax.numpy as jnp
from jax import lax
from jax.experimental import pallas as pl
from jax.experimental.pallas import tpu as pltpu
from jax.experimental.pallas import tpu_sc as plsc

EMBED_DIM = 32
NUM_CORES = 2
NUM_SUBCORES = 16
NUM_WORKERS = NUM_CORES * NUM_SUBCORES
CHUNK = 1024
TBL_BLK = 65536
OUT_BLK = 16384


def _idx_body(x_ref, o_ref):
    j = pl.program_id(0)
    x = x_ref[pl.ds(j, 1), :][0]
    tb4 = TBL_BLK // 4
    sh = tb4.bit_length() - 1
    o_ref[...] = (x & ~(TBL_BLK - 1)) + ((x & (tb4 - 1)) << 2) + (
        (x >> sh) & 3)


def _tbl_t_body(x_ref, o_ref):
    x = x_ref[...]
    b4 = TBL_BLK // 4
    y = jnp.concatenate([x[:, m * b4:(m + 1) * b4] for m in range(4)], axis=0)
    o_ref[...] = y.T


def _out_t_body(x_ref, o_ref):
    xt = x_ref[...].T
    ng = OUT_BLK // 2048
    o_ref[...] = jnp.concatenate(
        [xt[k * EMBED_DIM:(k + 1) * EMBED_DIM, g * 512:(g + 1) * 512]
         for g in range(ng) for k in range(4)], axis=1)


def _sc_gather(table_rm, idx_flat, n):
    per_worker = n // NUM_WORKERS
    n_chunks = per_worker // CHUNK
    half = CHUNK // 2
    mesh = plsc.VectorSubcoreMesh(core_axis_name="c", subcore_axis_name="s")

    @functools.partial(
        pl.kernel,
        out_type=jax.ShapeDtypeStruct((n // 4, 4 * EMBED_DIM),
                                      table_rm.dtype),
        mesh=mesh,
        scratch_types=[
            pltpu.VMEM((CHUNK,), jnp.int32),
            pltpu.VMEM((CHUNK,), jnp.int32),
            pltpu.VMEM((CHUNK, EMBED_DIM), jnp.float32),
            pltpu.VMEM((CHUNK, EMBED_DIM), jnp.float32),
            pltpu.SemaphoreType.DMA,
            pltpu.SemaphoreType.DMA,
            pltpu.SemaphoreType.DMA,
            pltpu.SemaphoreType.DMA,
        ],
        compiler_params=pltpu.CompilerParams(use_tc_tiling_on_sc=False),
    )
    def gather_kernel(tbl_hbm, idx_hbm, out_hbm,
                      idx_v0, idx_v1, rows_v0, rows_v1,
                      gsem0, gsem1, osem0, osem1):
        wid = lax.axis_index("s") * NUM_CORES + lax.axis_index("c")
        base = wid * per_worker
        idx_v = (idx_v0, idx_v1)
        rows_v = (rows_v0, rows_v1)
        gsem = (gsem0, gsem1)
        osem = (osem0, osem1)

        def start_gather(c):
            b = c % 2
            pltpu.sync_copy(idx_hbm.at[pl.ds(base + c * CHUNK, CHUNK)], idx_v[b])
            return pltpu.async_copy(tbl_hbm.at[idx_v[b]], rows_v[b], gsem[b])

        def start_drain(c):
            b = c % 2
            copies = []
            for h in range(2):
                p0 = base + c * CHUNK + h * half
                g = p0 // (4 * half)
                u = (p0 // half) % 4
                copies.append(pltpu.async_copy(
                    rows_v[b].at[pl.ds(h * half, half)],
                    out_hbm.at[pl.ds(g * half, half),
                               pl.ds(u * EMBED_DIM, EMBED_DIM)],
                    osem[b]))
            return copies

        gathers = [None, None]
        outs = [None, None]
        gathers[0] = start_gather(0)
        for c in range(n_chunks):
            b = c % 2
            nb = 1 - b
            if c + 1 < n_chunks:
                if outs[nb] is not None:
                    for o in outs[nb]:
                        o.wait()
                gathers[nb] = start_gather(c + 1)
            gathers[b].wait()
            outs[b] = start_drain(c)
        for pair in outs:
            if pair is not None:
                for o in pair:
                    o.wait()

    return gather_kernel(table_rm, idx_flat)


def kernel(cat_idx, table):
    batch, seq = cat_idx.shape
    n = batch * seq
    nv = table.shape[0]
    tb4 = TBL_BLK // 4
    n_ob = batch // OUT_BLK
    half_seq = seq // 2

    idx_t = jnp.swapaxes(cat_idx, 0, 1).astype(jnp.int32)
    idx_flat = pl.pallas_call(
        _idx_body,
        grid=(seq,),
        in_specs=[pl.BlockSpec((seq, batch), lambda j: (0, 0))],
        out_specs=pl.BlockSpec((batch,), lambda j: (j,)),
        out_shape=jax.ShapeDtypeStruct((n,), jnp.int32),
        compiler_params=pltpu.CompilerParams(
            dimension_semantics=("arbitrary",)),
    )(idx_t)

    table_t = jnp.swapaxes(table, 0, 1)
    n_tb = (nv + TBL_BLK - 1) // TBL_BLK
    nv_pad = n_tb * TBL_BLK
    table_rm4 = pl.pallas_call(
        _tbl_t_body,
        grid=(n_tb,),
        in_specs=[pl.BlockSpec((EMBED_DIM, TBL_BLK), lambda k: (0, k))],
        out_specs=pl.BlockSpec((tb4, 4 * EMBED_DIM), lambda k: (k, 0)),
        out_shape=jax.ShapeDtypeStruct((nv_pad // 4, 4 * EMBED_DIM),
                                       table.dtype),
        compiler_params=pltpu.CompilerParams(
            dimension_semantics=("parallel",)),
    )(table_t)
    table_rm = table_rm4.reshape(nv_pad, EMBED_DIM)

    lin128 = _sc_gather(table_rm, idx_flat, n)

    phys2d = pl.pallas_call(
        _out_t_body,
        grid=(seq, n_ob),
        in_specs=[pl.BlockSpec(
            (OUT_BLK // 4, 4 * EMBED_DIM),
            lambda j, i: (j * n_ob + i, 0))],
        out_specs=pl.BlockSpec((EMBED_DIM, OUT_BLK), lambda j, i: (j, i)),
        out_shape=jax.ShapeDtypeStruct((seq * EMBED_DIM, batch), table.dtype),
        compiler_params=pltpu.CompilerParams(
            dimension_semantics=("parallel", "parallel")),
    )(lin128)

    phys = phys2d.reshape(seq, EMBED_DIM, batch)
    return jnp.transpose(phys, (2, 0, 1))
